# Initial kernel scaffold; baseline (speedup 1.0000x reference)
import jax
import jax.numpy as jnp
from jax import lax
from jax.experimental import pallas as pl
from jax.experimental.pallas import tpu as pltpu
from jax.experimental.pallas import tpu_sc as plsc

N_NODES = 10000
N_EDGES = 320000
D = 128
NC, NS, LANES = 2, 16, 16
NW = NC * NS
EPW = N_EDGES // NW
CH = 125
NCHUNK = EPW // CH
NBUF = 4
NGROUP = NCHUNK // NBUF
RPT = N_NODES // NS

CF_SLICES = 10
CF_EPW = N_EDGES // CF_SLICES
CF_STAGE = 4
CF_CHUNK = CF_EPW // CF_STAGE
CF_BINS = N_NODES * 8
ZCH = 16000


def _msg_body(nodes_hbm, send_hbm, recv_hbm, zeros_hbm, out_hbm,
              sidx, ridx, rows, acc, gsem):
    c = lax.axis_index("c")
    s = lax.axis_index("s")
    wid = s * NC + c
    pltpu.sync_copy(send_hbm.at[wid], sidx)
    pltpu.sync_copy(recv_hbm.at[wid], ridx)
    pltpu.sync_copy(zeros_hbm, rows.at[0])
    base = s * RPT
    for k in range(RPT // CH):
        pltpu.sync_copy(rows.at[0], acc.at[pl.ds(base + k * CH, CH)])
    plsc.subcore_barrier()

    def group(g, carry):
        descs = []
        for b in range(NBUF):
            j = g * NBUF + b
            descs.append(pltpu.async_copy(nodes_hbm.at[sidx.at[j]], rows.at[b], gsem))
        for d in descs:
            d.wait()
        for b in range(NBUF):
            j = g * NBUF + b
            pltpu.sync_copy(rows.at[b], acc.at[ridx.at[j]], add=True)
        return carry

    lax.fori_loop(0, NGROUP, group, 0)
    plsc.subcore_barrier()
    pltpu.sync_copy(acc.at[pl.ds(base, RPT)], out_hbm.at[c, pl.ds(base, RPT)])


def _cnt_body(recv_hbm, val_hbm, zeros_hbm, out_hbm, cnt, rbuf, vbuf):
    c = lax.axis_index("c")
    s = lax.axis_index("s")
    wid = s * NC + c

    @pl.when(wid < 3 * CF_SLICES)
    def _():
        f = wid // CF_SLICES
        sl = wid % CF_SLICES
        for k in range(CF_BINS // ZCH):
            pltpu.sync_copy(zeros_hbm, cnt.at[pl.ds(k * ZCH, ZCH)])
        ones = jnp.ones((LANES,), jnp.float32)
        for t in range(CF_STAGE):
            pltpu.sync_copy(recv_hbm.at[sl, t], rbuf)
            pltpu.sync_copy(val_hbm.at[f, sl, t], vbuf)

            def body(i, carry):
                r = rbuf[pl.ds(i * LANES, LANES)]
                v = vbuf[pl.ds(i * LANES, LANES)]
                plsc.addupdate_scatter(cnt, [r * 8 + v], ones)
                return carry

            lax.fori_loop(0, CF_CHUNK // LANES, body, 0)
        pltpu.sync_copy(cnt, out_hbm.at[wid])


def _mlp_body(eps_ref, nodes_ref, msg_ref, cnt_ref, emb_ref,
              w1_ref, b1_ref, w2_ref, b2_ref, out_ref):
    x = (1.0 + eps_ref[0]) * nodes_ref[...] + msg_ref[0] + msg_ref[1]
    cnt = cnt_ref[...]
    for f in range(3):
        e = cnt[f * CF_SLICES]
        for w in range(1, CF_SLICES):
            e = e + cnt[f * CF_SLICES + w]
        x = x + jnp.dot(e, emb_ref[f], preferred_element_type=jnp.float32)
    h = jnp.maximum(jnp.dot(x, w1_ref[...], preferred_element_type=jnp.float32)
                    + b1_ref[...], 0.0)
    out_ref[...] = (jnp.dot(h, w2_ref[...], preferred_element_type=jnp.float32)
                    + b2_ref[...])


def _sc_mesh():
    return plsc.VectorSubcoreMesh(core_axis_name="c", subcore_axis_name="s",
                                  num_cores=NC, num_subcores=NS)


def _msg_call(nodes, send_r, recv_r, zeros2d):
    fn = pl.kernel(
        _msg_body,
        out_type=jax.ShapeDtypeStruct((NC, N_NODES, D), jnp.float32),
        mesh=_sc_mesh(),
        scratch_types=[
            pltpu.VMEM((NCHUNK, CH), jnp.int32),
            pltpu.VMEM((NCHUNK, CH), jnp.int32),
            pltpu.VMEM((NBUF, CH, D), jnp.float32),
            pltpu.VMEM_SHARED((N_NODES, D), jnp.float32),
            pltpu.SemaphoreType.DMA,
        ],
    )
    return fn(nodes, send_r, recv_r, zeros2d)


def _cnt_call(recv_c, val_c, zeros1d):
    fn = pl.kernel(
        _cnt_body,
        out_type=jax.ShapeDtypeStruct((3 * CF_SLICES, CF_BINS), jnp.float32),
        mesh=_sc_mesh(),
        scratch_types=[
            pltpu.VMEM((CF_BINS,), jnp.float32),
            pltpu.VMEM((CF_CHUNK,), jnp.int32),
            pltpu.VMEM((CF_CHUNK,), jnp.int32),
        ],
    )
    return fn(recv_c, val_c, zeros1d)


BT = 1000


def _mlp_call(epsilon, nodes, msg, cnts, embstack, W1, b1, W2, b2):
    return pl.pallas_call(
        _mlp_body,
        grid=(N_NODES // BT,),
        in_specs=[
            pl.BlockSpec(memory_space=pltpu.SMEM),
            pl.BlockSpec((BT, D), lambda i: (i, 0)),
            pl.BlockSpec((NC, BT, D), lambda i: (0, i, 0)),
            pl.BlockSpec((3 * CF_SLICES, BT, 8), lambda i: (0, i, 0)),
            pl.BlockSpec((3, 8, D), lambda i: (0, 0, 0)),
            pl.BlockSpec((D, D), lambda i: (0, 0)),
            pl.BlockSpec((1, D), lambda i: (0, 0)),
            pl.BlockSpec((D, D), lambda i: (0, 0)),
            pl.BlockSpec((1, D), lambda i: (0, 0)),
        ],
        out_specs=pl.BlockSpec((BT, D), lambda i: (i, 0)),
        out_shape=jax.ShapeDtypeStruct((N_NODES, D), jnp.float32),
    )(epsilon, nodes, msg, cnts, embstack, W1, b1, W2, b2)


def kernel(nodes, edges, receivers, senders, global_latent, node_graph_idx,
           edge_graph_idx, epsilon, W1, b1, W2, b2, emb0, emb1, emb2):
    send_r = senders.reshape(NW, NCHUNK, CH)
    recv_r = receivers.reshape(NW, NCHUNK, CH)
    zeros2d = jnp.zeros((CH, D), jnp.float32)
    msg = _msg_call(nodes, send_r, recv_r, zeros2d)

    recv_c = receivers.reshape(CF_SLICES, CF_STAGE, CF_CHUNK)
    val_c = edges.T.reshape(3, CF_SLICES, CF_STAGE, CF_CHUNK)
    zeros1d = jnp.zeros((ZCH,), jnp.float32)
    cnts = _cnt_call(recv_c, val_c, zeros1d).reshape(3 * CF_SLICES, N_NODES, 8)

    embstack = jnp.stack([emb0, emb1, emb2])
    node_update = _mlp_call(epsilon, nodes, msg, cnts, embstack,
                            W1, b1.reshape(1, D), W2, b2.reshape(1, D))
    return (node_update, edges, receivers, senders, global_latent,
            node_graph_idx, edge_graph_idx)


# speedup vs baseline: 8.3313x; 8.3313x over previous
import jax
import jax.numpy as jnp
from jax import lax
from jax.experimental import pallas as pl
from jax.experimental.pallas import tpu as pltpu
from jax.experimental.pallas import tpu_sc as plsc

N_NODES = 10000
N_EDGES = 320000
D = 128
NC, NS, LANES = 2, 16, 16
NW = NC * NS
HD = D // NC
EPW = N_EDGES // NS
CH = 125
NCHUNK = EPW // CH
NBUF = 4
NGROUP = NCHUNK // NBUF
RPT = N_NODES // NS

CF_SLICES = 10
CF_EPW = N_EDGES // CF_SLICES
CF_STAGE = 4
CF_CHUNK = CF_EPW // CF_STAGE
CF_BINS = N_NODES * 8
ZCH = 16000


def _msg_body(nodes_hbm, send_hbm, recv_hbm, zeros_hbm, out_hbm,
              sidx, ridx, rows, acc, gsem):
    c = lax.axis_index("c")
    s = lax.axis_index("s")
    pltpu.sync_copy(send_hbm.at[c, s], sidx)
    pltpu.sync_copy(recv_hbm.at[s], ridx)
    pltpu.sync_copy(zeros_hbm, rows.at[0])
    base = s * RPT
    for k in range(RPT // CH):
        pltpu.sync_copy(rows.at[0], acc.at[pl.ds(base + k * CH, CH)])
    plsc.subcore_barrier()

    def group(g, carry):
        descs = []
        for b in range(NBUF):
            j = g * NBUF + b
            descs.append(pltpu.async_copy(nodes_hbm.at[sidx.at[j]], rows.at[b], gsem))
        for d in descs:
            d.wait()
        for b in range(NBUF):
            j = g * NBUF + b
            pltpu.sync_copy(rows.at[b], acc.at[ridx.at[j]], add=True)
        return carry

    lax.fori_loop(0, NGROUP, group, 0)
    plsc.subcore_barrier()
    pltpu.sync_copy(acc.at[pl.ds(base, RPT)], out_hbm.at[c, pl.ds(base, RPT)])


def _cnt_body(recv_hbm, val_hbm, zeros_hbm, out_hbm, cnt, rbuf, vbuf):
    c = lax.axis_index("c")
    s = lax.axis_index("s")
    wid = s * NC + c

    @pl.when(wid < 3 * CF_SLICES)
    def _():
        f = wid // CF_SLICES
        sl = wid % CF_SLICES
        for k in range(CF_BINS // ZCH):
            pltpu.sync_copy(zeros_hbm, cnt.at[pl.ds(k * ZCH, ZCH)])
        ones = jnp.ones((LANES,), jnp.float32)
        for t in range(CF_STAGE):
            pltpu.sync_copy(recv_hbm.at[sl, t], rbuf)
            pltpu.sync_copy(val_hbm.at[f, sl, t], vbuf)

            def body(i, carry):
                r = rbuf[pl.ds(i * LANES, LANES)]
                v = vbuf[pl.ds(i * LANES, LANES)]
                plsc.addupdate_scatter(cnt, [r * 8 + v], ones)
                return carry

            lax.fori_loop(0, CF_CHUNK // LANES, body, 0)
        pltpu.sync_copy(cnt, out_hbm.at[wid])


def _mlp_body(eps_ref, nodes_ref, msg_ref, cnt_ref, emb_ref,
              w1_ref, b1_ref, w2_ref, b2_ref, out_ref):
    x = (1.0 + eps_ref[0]) * nodes_ref[...]
    x = x + jnp.concatenate([msg_ref[0], msg_ref[1]], axis=-1)
    cnt = cnt_ref[...]
    for f in range(3):
        e = cnt[f * CF_SLICES]
        for w in range(1, CF_SLICES):
            e = e + cnt[f * CF_SLICES + w]
        x = x + jnp.dot(e, emb_ref[f], preferred_element_type=jnp.float32)
    h = jnp.maximum(jnp.dot(x, w1_ref[...], preferred_element_type=jnp.float32)
                    + b1_ref[...], 0.0)
    out_ref[...] = (jnp.dot(h, w2_ref[...], preferred_element_type=jnp.float32)
                    + b2_ref[...])


def _sc_mesh():
    return plsc.VectorSubcoreMesh(core_axis_name="c", subcore_axis_name="s",
                                  num_cores=NC, num_subcores=NS)


def _msg_call(nodes_stack, send_r, recv_r, zeros2d):
    fn = pl.kernel(
        _msg_body,
        out_type=jax.ShapeDtypeStruct((NC, N_NODES, HD), jnp.float32),
        mesh=_sc_mesh(),
        scratch_types=[
            pltpu.VMEM((NCHUNK, CH), jnp.int32),
            pltpu.VMEM((NCHUNK, CH), jnp.int32),
            pltpu.VMEM((NBUF, CH, HD), jnp.float32),
            pltpu.VMEM_SHARED((N_NODES, HD), jnp.float32),
            pltpu.SemaphoreType.DMA,
        ],
        compiler_params=pltpu.CompilerParams(needs_layout_passes=False,
                                             use_tc_tiling_on_sc=False),
    )
    return fn(nodes_stack, send_r, recv_r, zeros2d)


def _cnt_call(recv_c, val_c, zeros1d):
    fn = pl.kernel(
        _cnt_body,
        out_type=jax.ShapeDtypeStruct((3 * CF_SLICES, CF_BINS), jnp.float32),
        mesh=_sc_mesh(),
        scratch_types=[
            pltpu.VMEM((CF_BINS,), jnp.float32),
            pltpu.VMEM((CF_CHUNK,), jnp.int32),
            pltpu.VMEM((CF_CHUNK,), jnp.int32),
        ],
        compiler_params=pltpu.CompilerParams(needs_layout_passes=False),
    )
    return fn(recv_c, val_c, zeros1d)


BT = 1000


def _mlp_call(epsilon, nodes, msg, cnts, embstack, W1, b1, W2, b2):
    return pl.pallas_call(
        _mlp_body,
        grid=(N_NODES // BT,),
        in_specs=[
            pl.BlockSpec(memory_space=pltpu.SMEM),
            pl.BlockSpec((BT, D), lambda i: (i, 0)),
            pl.BlockSpec((NC, BT, HD), lambda i: (0, i, 0)),
            pl.BlockSpec((3 * CF_SLICES, BT, 8), lambda i: (0, i, 0)),
            pl.BlockSpec((3, 8, D), lambda i: (0, 0, 0)),
            pl.BlockSpec((D, D), lambda i: (0, 0)),
            pl.BlockSpec((1, D), lambda i: (0, 0)),
            pl.BlockSpec((D, D), lambda i: (0, 0)),
            pl.BlockSpec((1, D), lambda i: (0, 0)),
        ],
        out_specs=pl.BlockSpec((BT, D), lambda i: (i, 0)),
        out_shape=jax.ShapeDtypeStruct((N_NODES, D), jnp.float32),
    )(epsilon, nodes, msg, cnts, embstack, W1, b1, W2, b2)


def kernel(nodes, edges, receivers, senders, global_latent, node_graph_idx,
           edge_graph_idx, epsilon, W1, b1, W2, b2, emb0, emb1, emb2):
    send_base = senders.reshape(NS, NCHUNK, CH)
    send_r = jnp.stack([send_base, send_base + N_NODES])
    recv_r = receivers.reshape(NS, NCHUNK, CH)
    nodes_stack = jnp.concatenate([nodes[:, :HD], nodes[:, HD:]], axis=0)
    zeros2d = jnp.zeros((CH, HD), jnp.float32)
    msg = _msg_call(nodes_stack, send_r, recv_r, zeros2d)

    recv_c = receivers.reshape(CF_SLICES, CF_STAGE, CF_CHUNK)
    val_c = edges.T.reshape(3, CF_SLICES, CF_STAGE, CF_CHUNK)
    zeros1d = jnp.zeros((ZCH,), jnp.float32)
    cnts = _cnt_call(recv_c, val_c, zeros1d).reshape(3 * CF_SLICES, N_NODES, 8)

    embstack = jnp.stack([emb0, emb1, emb2])
    node_update = _mlp_call(epsilon, nodes, msg, cnts, embstack,
                            W1, b1.reshape(1, D), W2, b2.reshape(1, D))
    return (node_update, edges, receivers, senders, global_latent,
            node_graph_idx, edge_graph_idx)
